# Initial kernel scaffold; baseline (speedup 1.0000x reference)
import jax
import jax.numpy as jnp
from jax.experimental import pallas as pl


def kernel(seg, byx, gfeat):
    raise NotImplementedError("write your pallas kernel here")


# speedup vs baseline: 126.1779x; 126.1779x over previous
import jax
import jax.numpy as jnp
from jax.experimental import pallas as pl
from jax.experimental.pallas import tpu as pltpu
from functools import partial

_B, _H, _W = 4, 224, 224
_NV = 1024
_P = 16
_SIGMA = 0.05
_N = _B * _H * _W
_C = 2048
_NBLK = _N // _C


def _soft_w_t(v_row, grid_col, inv2s2):
    d = v_row - grid_col
    w = jnp.exp(-(d * d) * inv2s2)
    s = jnp.sum(w, axis=0, keepdims=True)
    return w / (s + 1e-12)


def _hist_kernel(seg_ref, y_ref, x_ref, g1_ref, g2_ref, out_ref, acc_ref):
    i = pl.program_id(0)
    gi = jax.lax.broadcasted_iota(jnp.int32, (_P, 1), 0)
    grid_col = gi.astype(jnp.float32) * (2.0 / (_P - 1)) - 1.0
    inv2s2 = 1.0 / (2.0 * _SIGMA * _SIGMA)

    v0 = y_ref[0] * (2.0 / _H) - 1.0
    v1 = x_ref[0] * (2.0 / _W) - 1.0
    v2 = g1_ref[0]
    v3 = g2_ref[0]

    wa0 = _soft_w_t(v0, grid_col, inv2s2).astype(jnp.bfloat16)
    wb0 = _soft_w_t(v1, grid_col, inv2s2).astype(jnp.bfloat16)
    wa1 = _soft_w_t(v2, grid_col, inv2s2).astype(jnp.bfloat16)
    wb1 = _soft_w_t(v3, grid_col, inv2s2).astype(jnp.bfloat16)

    j0 = (wa0[:, None, :] * wb0[None, :, :]).reshape(_P * _P, _C)
    j1 = (wa1[:, None, :] * wb1[None, :, :]).reshape(_P * _P, _C)
    j = jnp.concatenate([j0, j1], axis=0)

    seg_col = seg_ref[0]
    iota_row = jax.lax.broadcasted_iota(jnp.int32, (1, _NV), 1)
    onehot = (seg_col == iota_row).astype(jnp.bfloat16)

    contrib = jnp.dot(j, onehot, preferred_element_type=jnp.float32)

    @pl.when(i == 0)
    def _():
        acc_ref[...] = jnp.zeros_like(acc_ref)

    acc_ref[...] += contrib

    @pl.when(i == _NBLK - 1)
    def _():
        acc = acc_ref[...]
        sizes = jnp.sum(acc[: _P * _P, :], axis=0, keepdims=True)
        out_ref[...] = acc * (4.0 / sizes)


def kernel(seg, byx, gfeat):
    seg_b = seg.reshape(-1).reshape(_NBLK, _C, 1)
    yf = byx[1].astype(jnp.float32).reshape(_NBLK, 1, _C)
    xf = byx[2].astype(jnp.float32).reshape(_NBLK, 1, _C)
    g1 = gfeat[:, 0, :, :].reshape(-1).reshape(_NBLK, 1, _C)
    g2 = gfeat[:, 1, :, :].reshape(-1).reshape(_NBLK, 1, _C)

    row_spec = pl.BlockSpec((1, 1, _C), lambda i: (i, 0, 0))
    out = pl.pallas_call(
        _hist_kernel,
        grid=(_NBLK,),
        in_specs=[
            pl.BlockSpec((1, _C, 1), lambda i: (i, 0, 0)),
            row_spec, row_spec, row_spec, row_spec,
        ],
        out_specs=pl.BlockSpec((2 * _P * _P, _NV), lambda i: (0, 0)),
        out_shape=jax.ShapeDtypeStruct((2 * _P * _P, _NV), jnp.float32),
        scratch_shapes=[pltpu.VMEM((2 * _P * _P, _NV), jnp.float32)],
        compiler_params=pltpu.CompilerParams(
            dimension_semantics=("arbitrary",),
        ),
    )(seg_b, yf, xf, g1, g2)

    return out.reshape(2, _P, _P, _NV).transpose(3, 0, 1, 2)


# speedup vs baseline: 135.0985x; 1.0707x over previous
import jax
import jax.numpy as jnp
from jax.experimental import pallas as pl
from jax.experimental.pallas import tpu as pltpu
from functools import partial

_B, _H, _W = 4, 224, 224
_NV = 1024
_P = 16
_SIGMA = 0.05
_N = _B * _H * _W
_C = 4096
_NBLK = _N // _C


def _soft_w_t(v_row, grid_col, inv2s2):
    d = v_row - grid_col
    w = jnp.exp(-(d * d) * inv2s2)
    s = jnp.sum(w, axis=0, keepdims=True)
    return w / (s + 1e-12)


def _hist_kernel(seg_ref, y_ref, x_ref, g1_ref, g2_ref, out_ref, acc_ref):
    i = pl.program_id(0)
    gi = jax.lax.broadcasted_iota(jnp.int32, (_P, 1), 0)
    grid_col = gi.astype(jnp.float32) * (2.0 / (_P - 1)) - 1.0
    inv2s2 = 1.0 / (2.0 * _SIGMA * _SIGMA)

    v0 = y_ref[0] * (2.0 / _H) - 1.0
    v1 = x_ref[0] * (2.0 / _W) - 1.0
    v2 = g1_ref[0]
    v3 = g2_ref[0]

    wa0 = _soft_w_t(v0, grid_col, inv2s2).astype(jnp.bfloat16)
    wb0 = _soft_w_t(v1, grid_col, inv2s2).astype(jnp.bfloat16)
    wa1 = _soft_w_t(v2, grid_col, inv2s2).astype(jnp.bfloat16)
    wb1 = _soft_w_t(v3, grid_col, inv2s2).astype(jnp.bfloat16)

    j0 = (wa0[:, None, :] * wb0[None, :, :]).reshape(_P * _P, _C)
    j1 = (wa1[:, None, :] * wb1[None, :, :]).reshape(_P * _P, _C)
    j = jnp.concatenate([j0, j1], axis=0)

    seg_col = seg_ref[0]
    iota_row = jax.lax.broadcasted_iota(jnp.int32, (1, _NV), 1)
    onehot = (seg_col == iota_row).astype(jnp.bfloat16)

    contrib = jnp.dot(j, onehot, preferred_element_type=jnp.float32)

    @pl.when(i == 0)
    def _():
        acc_ref[...] = jnp.zeros_like(acc_ref)

    acc_ref[...] += contrib

    @pl.when(i == _NBLK - 1)
    def _():
        acc = acc_ref[...]
        sizes = jnp.sum(acc[: _P * _P, :], axis=0, keepdims=True)
        out_ref[...] = acc * (4.0 / sizes)


def kernel(seg, byx, gfeat):
    seg_b = seg.reshape(-1).reshape(_NBLK, _C, 1)
    yf = byx[1].astype(jnp.float32).reshape(_NBLK, 1, _C)
    xf = byx[2].astype(jnp.float32).reshape(_NBLK, 1, _C)
    g1 = gfeat[:, 0, :, :].reshape(-1).reshape(_NBLK, 1, _C)
    g2 = gfeat[:, 1, :, :].reshape(-1).reshape(_NBLK, 1, _C)

    row_spec = pl.BlockSpec((1, 1, _C), lambda i: (i, 0, 0))
    out = pl.pallas_call(
        _hist_kernel,
        grid=(_NBLK,),
        in_specs=[
            pl.BlockSpec((1, _C, 1), lambda i: (i, 0, 0)),
            row_spec, row_spec, row_spec, row_spec,
        ],
        out_specs=pl.BlockSpec((2 * _P * _P, _NV), lambda i: (0, 0)),
        out_shape=jax.ShapeDtypeStruct((2 * _P * _P, _NV), jnp.float32),
        scratch_shapes=[pltpu.VMEM((2 * _P * _P, _NV), jnp.float32)],
        compiler_params=pltpu.CompilerParams(
            dimension_semantics=("arbitrary",),
        ),
    )(seg_b, yf, xf, g1, g2)

    return out.reshape(2, _P, _P, _NV).transpose(3, 0, 1, 2)
